# Initial kernel scaffold; baseline (speedup 1.0000x reference)
import jax
import jax.numpy as jnp
from jax.experimental import pallas as pl


def kernel(x, edge_index, W1, b1, W2, b2, W3, b3, W4, b4, Wo, bo):
    raise NotImplementedError("write your pallas kernel here")


# speedup vs baseline: 1.6321x; 1.6321x over previous
import functools

import jax
import jax.numpy as jnp
from jax import lax
from jax.experimental import pallas as pl
from jax.experimental.pallas import tpu as pltpu
from jax.experimental.pallas import tpu_sc as plsc

N_NODES = 10000
N_PAD = 10240
E_EDGES = 160000
E_PAD = 163840
NC, NS = 2, 16
WIN = 128
ROWS_PER_TILE = N_PAD // NS
EDGES_PER_W32 = E_PAD // (NC * NS)
EDGES_PER_W16 = E_PAD // NS

_MESH = dict(core_axis_name="c", subcore_axis_name="s", num_cores=NC,
             num_subcores=NS)


def _make_deg_kernel():
    mesh = plsc.VectorSubcoreMesh(**_MESH)

    @functools.partial(
        pl.kernel,
        out_type=jax.ShapeDtypeStruct((NC, N_PAD, 128), jnp.float32),
        mesh=mesh,
        scratch_types=[
            pltpu.VMEM_SHARED((N_PAD, 128), jnp.float32),
            pltpu.VMEM((WIN, 128), jnp.float32),
            pltpu.VMEM((WIN,), jnp.int32),
        ],
    )
    def deg_k(colp, ones_hbm, zeros_hbm, out, slab, ones_v, cidx):
        c = lax.axis_index("c")
        s = lax.axis_index("s")
        wid = c * NS + s
        pltpu.sync_copy(zeros_hbm, slab.at[pl.ds(s * ROWS_PER_TILE,
                                                 ROWS_PER_TILE)])
        pltpu.sync_copy(ones_hbm, ones_v)
        plsc.subcore_barrier()

        def body(w, carry):
            base = wid * EDGES_PER_W32 + w * WIN
            pltpu.sync_copy(colp.at[pl.ds(base, WIN)], cidx)
            pltpu.sync_copy(ones_v, slab.at[cidx], add=True)
            return carry

        lax.fori_loop(0, EDGES_PER_W32 // WIN, body, 0)
        plsc.subcore_barrier()
        pltpu.sync_copy(slab.at[pl.ds(s * ROWS_PER_TILE, ROWS_PER_TILE)],
                        out.at[c, pl.ds(s * ROWS_PER_TILE, ROWS_PER_TILE)])

    return deg_k


def _make_agg_kernel(K):
    mesh = plsc.VectorSubcoreMesh(**_MESH)
    cpc = K // NC

    @functools.partial(
        pl.kernel,
        out_type=jax.ShapeDtypeStruct((K, N_PAD, 128), jnp.float32),
        mesh=mesh,
        scratch_types=[
            pltpu.VMEM_SHARED((N_PAD, 128), jnp.float32),
            pltpu.VMEM((WIN, 128), jnp.float32),
            pltpu.VMEM((WIN,), jnp.int32),
            pltpu.VMEM((WIN,), jnp.int32),
            pltpu.VMEM((WIN,), jnp.int32),
            pltpu.SemaphoreType.DMA,
        ],
    )
    def agg_k(g2d, rowp, colp, zeros_hbm, out, slab, rows_v, ridx, gidx,
              cidx, sem):
        cid = lax.axis_index("c")
        s = lax.axis_index("s")
        for cl in range(cpc):
            ch = cid * cpc + cl
            pltpu.sync_copy(zeros_hbm, slab.at[pl.ds(s * ROWS_PER_TILE,
                                                     ROWS_PER_TILE)])
            plsc.subcore_barrier()

            def body(w, carry):
                base = s * EDGES_PER_W16 + w * WIN
                pltpu.sync_copy(rowp.at[pl.ds(base, WIN)], ridx)
                pltpu.sync_copy(colp.at[pl.ds(base, WIN)], cidx)
                for i in range(WIN // 16):
                    v = ridx[pl.ds(i * 16, 16)]
                    gidx[pl.ds(i * 16, 16)] = v * K + ch
                pltpu.async_copy(g2d.at[gidx], rows_v, sem).wait()
                pltpu.sync_copy(rows_v, slab.at[cidx], add=True)
                return carry

            lax.fori_loop(0, EDGES_PER_W16 // WIN, body, 0)
            plsc.subcore_barrier()
            pltpu.sync_copy(
                slab.at[pl.ds(s * ROWS_PER_TILE, ROWS_PER_TILE)],
                out.at[ch, pl.ds(s * ROWS_PER_TILE, ROWS_PER_TILE)])
            plsc.subcore_barrier()

    return agg_k


_BM = 256


def _prep_body(h0_ref, h1_ref, x_ref, dis_ref, g0_ref):
    deg128 = h0_ref[...] + h1_ref[...] + 1.0
    dis_ref[...] = lax.rsqrt(deg128)
    dis = lax.rsqrt(deg128[:, :1])
    g0_ref[...] = x_ref[...] * dis


def _prep(h0, h1, x_p):
    grid = (N_PAD // _BM,)
    return pl.pallas_call(
        _prep_body,
        grid=grid,
        in_specs=[
            pl.BlockSpec((_BM, 128), lambda i: (i, 0)),
            pl.BlockSpec((_BM, 128), lambda i: (i, 0)),
            pl.BlockSpec((_BM, 256), lambda i: (i, 0)),
        ],
        out_specs=[
            pl.BlockSpec((_BM, 128), lambda i: (i, 0)),
            pl.BlockSpec((_BM, 256), lambda i: (i, 0)),
        ],
        out_shape=[
            jax.ShapeDtypeStruct((N_PAD, 128), jnp.float32),
            jax.ShapeDtypeStruct((N_PAD, 256), jnp.float32),
        ],
    )(h0, h1, x_p)


def _mm_layer(S, g, dis, W, b, relu, scale_out, bn=512):
    K = S.shape[0]
    dout = W.shape[1]
    grid = (N_PAD // _BM, dout // bn, K)

    def body(s_ref, g_ref, dis_ref, w_ref, b_ref, out_ref):
        k = pl.program_id(2)
        nk = pl.num_programs(2)
        a = (s_ref[0] + g_ref[...]) * dis_ref[...]
        part = jnp.dot(a, w_ref[...], preferred_element_type=jnp.float32)

        @pl.when(k == 0)
        def _():
            out_ref[...] = part

        @pl.when(k > 0)
        def _():
            out_ref[...] += part

        @pl.when(k == nk - 1)
        def _():
            r = out_ref[...] + b_ref[...]
            if relu:
                r = jnp.maximum(r, 0.0)
            if scale_out:
                r = r * jnp.broadcast_to(dis_ref[:, :1], r.shape)
            out_ref[...] = r

    return pl.pallas_call(
        body,
        grid=grid,
        in_specs=[
            pl.BlockSpec((1, _BM, 128), lambda i, j, k: (k, i, 0)),
            pl.BlockSpec((_BM, 128), lambda i, j, k: (i, k)),
            pl.BlockSpec((_BM, 128), lambda i, j, k: (i, 0)),
            pl.BlockSpec((128, bn), lambda i, j, k: (k, j)),
            pl.BlockSpec((1, bn), lambda i, j, k: (0, j)),
        ],
        out_specs=pl.BlockSpec((_BM, bn), lambda i, j, k: (i, j)),
        out_shape=jax.ShapeDtypeStruct((N_PAD, dout), jnp.float32),
        compiler_params=pltpu.CompilerParams(
            dimension_semantics=("parallel", "parallel", "arbitrary")),
    )(S, g, dis, W, b)


def _mm_plain_scaled(A, dis, W, bm=_BM, bn=512, bk=512):
    din, dout = W.shape
    grid = (N_PAD // bm, dout // bn, din // bk)

    def body(a_ref, dis_ref, w_ref, out_ref):
        k = pl.program_id(2)
        nk = pl.num_programs(2)
        part = jnp.dot(a_ref[...], w_ref[...],
                       preferred_element_type=jnp.float32)

        @pl.when(k == 0)
        def _():
            out_ref[...] = part

        @pl.when(k > 0)
        def _():
            out_ref[...] += part

        @pl.when(k == nk - 1)
        def _():
            out_ref[...] = out_ref[...] * jnp.broadcast_to(
                dis_ref[:, :1], out_ref.shape)

    return pl.pallas_call(
        body,
        grid=grid,
        in_specs=[
            pl.BlockSpec((bm, bk), lambda i, j, k: (i, k)),
            pl.BlockSpec((bm, 128), lambda i, j, k: (i, 0)),
            pl.BlockSpec((bk, bn), lambda i, j, k: (k, j)),
        ],
        out_specs=pl.BlockSpec((bm, bn), lambda i, j, k: (i, j)),
        out_shape=jax.ShapeDtypeStruct((N_PAD, dout), jnp.float32),
        compiler_params=pltpu.CompilerParams(
            dimension_semantics=("parallel", "parallel", "arbitrary")),
    )(A, dis, W)


def _mm_final(S, g, dis, b4, Wo, bo):
    K = S.shape[0]
    grid = (N_PAD // _BM, K)

    def body(s_ref, g_ref, dis_ref, b4_ref, wo_ref, bo_ref, out_ref):
        k = pl.program_id(1)
        nk = pl.num_programs(1)
        h = (s_ref[0] + g_ref[...]) * dis_ref[...] + b4_ref[...]
        h = jnp.maximum(h, 0.0)
        part = jnp.dot(h, wo_ref[...], preferred_element_type=jnp.float32)

        @pl.when(k == 0)
        def _():
            out_ref[...] = part

        @pl.when(k > 0)
        def _():
            out_ref[...] += part

        @pl.when(k == nk - 1)
        def _():
            out_ref[...] += bo_ref[...]

    return pl.pallas_call(
        body,
        grid=grid,
        in_specs=[
            pl.BlockSpec((1, _BM, 128), lambda i, k: (k, i, 0)),
            pl.BlockSpec((_BM, 128), lambda i, k: (i, k)),
            pl.BlockSpec((_BM, 128), lambda i, k: (i, 0)),
            pl.BlockSpec((1, 128), lambda i, k: (0, k)),
            pl.BlockSpec((128, 128), lambda i, k: (k, 0)),
            pl.BlockSpec((1, 128), lambda i, k: (0, 0)),
        ],
        out_specs=pl.BlockSpec((_BM, 128), lambda i, k: (i, 0)),
        out_shape=jax.ShapeDtypeStruct((N_PAD, 128), jnp.float32),
        compiler_params=pltpu.CompilerParams(
            dimension_semantics=("parallel", "arbitrary")),
    )(S, g, dis, b4, Wo, bo)


_deg_kernel = _make_deg_kernel()
_agg_kernels = {K: _make_agg_kernel(K) for K in (2, 8, 16)}


def kernel(x, edge_index, W1, b1, W2, b2, W3, b3, W4, b4, Wo, bo):
    row = edge_index[0].astype(jnp.int32)
    col = edge_index[1].astype(jnp.int32)
    npad = E_PAD - E_EDGES
    pad_i = jnp.arange(npad, dtype=jnp.int32)
    rowp = jnp.concatenate([row, pad_i % N_NODES])
    colp = jnp.concatenate([col, N_NODES + pad_i % (N_PAD - N_NODES)])

    ones128 = jnp.ones((WIN, 128), jnp.float32)
    zeros128 = jnp.zeros((ROWS_PER_TILE, 128), jnp.float32)

    hist = _deg_kernel(colp, ones128, zeros128)
    x_p = jnp.pad(x, ((0, N_PAD - N_NODES), (0, 0)))
    dis, g0 = _prep(hist[0], hist[1], x_p)

    def agg(g, K):
        return _agg_kernels[K](g.reshape(N_PAD * K, 128), rowp, colp,
                               zeros128)

    S0 = agg(g0, 2)
    g1 = _mm_layer(S0, g0, dis, W1, b1.reshape(1, -1), relu=True,
                   scale_out=True)
    S1 = agg(g1, 8)
    g2 = _mm_layer(S1, g1, dis, W2, b2.reshape(1, -1), relu=True,
                   scale_out=True)
    S2 = agg(g2, 16)
    h3 = _mm_layer(S2, g2, dis, W3, b3.reshape(1, -1), relu=True,
                   scale_out=False)
    g3 = _mm_plain_scaled(h3, dis, W4)
    S3 = agg(g3, 16)
    out = _mm_final(S3, g3, dis, b4.reshape(1, -1), Wo, bo.reshape(1, -1))
    return out[:N_NODES]


# speedup vs baseline: 2.1665x; 1.3274x over previous
import functools

import jax
import jax.numpy as jnp
from jax import lax
from jax.experimental import pallas as pl
from jax.experimental.pallas import tpu as pltpu
from jax.experimental.pallas import tpu_sc as plsc

N_NODES = 10000
N_PAD = 10240
E_EDGES = 160000
E_PAD = 163840
NC, NS = 2, 16
WIN = 128
ROWS_PER_TILE = N_PAD // NS
EDGES_PER_W32 = E_PAD // (NC * NS)
EDGES_PER_W16 = E_PAD // NS

_MESH = dict(core_axis_name="c", subcore_axis_name="s", num_cores=NC,
             num_subcores=NS)


def _make_deg_kernel():
    mesh = plsc.VectorSubcoreMesh(**_MESH)

    @functools.partial(
        pl.kernel,
        out_type=jax.ShapeDtypeStruct((NC, N_PAD, 128), jnp.float32),
        mesh=mesh,
        scratch_types=[
            pltpu.VMEM_SHARED((N_PAD, 128), jnp.float32),
            pltpu.VMEM((WIN, 128), jnp.float32),
            pltpu.VMEM((WIN,), jnp.int32),
        ],
    )
    def deg_k(colp, ones_hbm, zeros_hbm, out, slab, ones_v, cidx):
        c = lax.axis_index("c")
        s = lax.axis_index("s")
        wid = c * NS + s
        pltpu.sync_copy(zeros_hbm, slab.at[pl.ds(s * ROWS_PER_TILE,
                                                 ROWS_PER_TILE)])
        pltpu.sync_copy(ones_hbm, ones_v)
        plsc.subcore_barrier()

        def body(w, carry):
            base = wid * EDGES_PER_W32 + w * WIN
            pltpu.sync_copy(colp.at[pl.ds(base, WIN)], cidx)
            pltpu.sync_copy(ones_v, slab.at[cidx], add=True)
            return carry

        lax.fori_loop(0, EDGES_PER_W32 // WIN, body, 0)
        plsc.subcore_barrier()
        pltpu.sync_copy(slab.at[pl.ds(s * ROWS_PER_TILE, ROWS_PER_TILE)],
                        out.at[c, pl.ds(s * ROWS_PER_TILE, ROWS_PER_TILE)])

    return deg_k


NWIN = EDGES_PER_W16 // WIN
NWIN_H = NWIN // 2
SLAB_ROWS = N_PAD
TROWS = SLAB_ROWS // NS


def _make_agg_kernel(K):
    mesh = plsc.VectorSubcoreMesh(**_MESH)
    cpc = K // NC

    @functools.partial(
        pl.kernel,
        out_type=jax.ShapeDtypeStruct((K, N_PAD, 128), jnp.float32),
        mesh=mesh,
        scratch_types=[
            pltpu.VMEM_SHARED((SLAB_ROWS, 128), jnp.float32),
            pltpu.VMEM((WIN, 128), jnp.float32),
            pltpu.VMEM((WIN, 128), jnp.float32),
            [pltpu.VMEM((WIN,), jnp.int32)] * 4,
            [pltpu.VMEM((WIN,), jnp.int32)] * 4,
            [pltpu.SemaphoreType.DMA] * 4,
            [pltpu.SemaphoreType.DMA] * 2,
            [pltpu.SemaphoreType.DMA] * 2,
        ],
    )
    def agg_k(g2df, gidxf, colf, zeros_hbm, out, slab, rows0, rows1,
              gbuf, cbuf, isem, gsem, ssem):
        cid = lax.axis_index("c")
        s = lax.axis_index("s")
        rows = (rows0, rows1)

        def idx_start(w, q, ch):
            gbase = (ch * NS + s) * EDGES_PER_W16 + w * WIN
            cbase = s * EDGES_PER_W16 + w * WIN
            pltpu.async_copy(gidxf.at[pl.ds(gbase, WIN)], gbuf[q],
                             isem[q])
            pltpu.async_copy(colf.at[pl.ds(cbase, WIN)], cbuf[q],
                             isem[q])

        def idx_wait(q):
            pltpu.make_async_copy(colf.at[pl.ds(0, WIN)], gbuf[q],
                                  isem[q]).wait()
            pltpu.make_async_copy(colf.at[pl.ds(0, WIN)], cbuf[q],
                                  isem[q]).wait()

        def gather_start(q, p):
            pltpu.async_copy(g2df.at[gbuf[q]], rows[p], gsem[p])

        def gather_wait(q, p):
            pltpu.make_async_copy(g2df.at[gbuf[q]], rows[p],
                                  gsem[p]).wait()

        def scatter_start(q, p):
            pltpu.async_copy(rows[p], slab.at[cbuf[q]], ssem[p],
                             add=True)

        def scatter_wait(q, p):
            pltpu.make_async_copy(rows[p], slab.at[cbuf[q]],
                                  ssem[p]).wait()

        def step(w, q, ch, do_idx, do_gather):
            p = q % 2
            gather_wait(q, p)
            scatter_start(q, p)
            scatter_wait(q, p)
            if do_idx:
                idx_start(w + 4, q, ch)
            if do_gather:
                q2 = (q + 2) % 4
                idx_wait(q2)
                gather_start(q2, p)

        for cl in range(cpc):
            ch = cid * cpc + cl
            pltpu.sync_copy(zeros_hbm, slab.at[pl.ds(s * TROWS, TROWS)])
            plsc.subcore_barrier()

            for w in range(4):
                idx_start(w, w, ch)
            for w in range(2):
                idx_wait(w)
                gather_start(w, w)

            def body(j, carry):
                for qq in range(4):
                    step(4 * j + qq, qq, ch, True, True)
                return carry

            lax.fori_loop(0, NWIN // 4 - 2, body, 0)
            for w in range(NWIN - 8, NWIN):
                step(w, w % 4, ch, w + 4 < NWIN, w + 2 < NWIN)
            plsc.subcore_barrier()
            pltpu.sync_copy(
                slab.at[pl.ds(s * TROWS, TROWS)],
                out.at[ch, pl.ds(s * TROWS, TROWS)])
            plsc.subcore_barrier()

    return agg_k


_BM = 256


def _prep_body(h0_ref, h1_ref, x_ref, dis_ref, g0_ref):
    deg128 = h0_ref[...] + h1_ref[...] + 1.0
    dis_ref[...] = lax.rsqrt(deg128)
    dis = lax.rsqrt(deg128[:, :1])
    g0_ref[...] = x_ref[...] * dis


def _prep(h0, h1, x_p):
    grid = (N_PAD // _BM,)
    return pl.pallas_call(
        _prep_body,
        grid=grid,
        in_specs=[
            pl.BlockSpec((_BM, 128), lambda i: (i, 0)),
            pl.BlockSpec((_BM, 128), lambda i: (i, 0)),
            pl.BlockSpec((_BM, 256), lambda i: (i, 0)),
        ],
        out_specs=[
            pl.BlockSpec((_BM, 128), lambda i: (i, 0)),
            pl.BlockSpec((_BM, 256), lambda i: (i, 0)),
        ],
        out_shape=[
            jax.ShapeDtypeStruct((N_PAD, 128), jnp.float32),
            jax.ShapeDtypeStruct((N_PAD, 256), jnp.float32),
        ],
    )(h0, h1, x_p)


def _mm_layer(S, g, dis, W, b, relu, scale_out, bn=512):
    K = S.shape[0]
    dout = W.shape[1]
    grid = (N_PAD // _BM, dout // bn, K)

    def body(s_ref, g_ref, dis_ref, w_ref, b_ref, out_ref):
        k = pl.program_id(2)
        nk = pl.num_programs(2)
        a = (s_ref[0] + g_ref[...]) * dis_ref[...]
        part = jnp.dot(a, w_ref[...], preferred_element_type=jnp.float32)

        @pl.when(k == 0)
        def _():
            out_ref[...] = part

        @pl.when(k > 0)
        def _():
            out_ref[...] += part

        @pl.when(k == nk - 1)
        def _():
            r = out_ref[...] + b_ref[...]
            if relu:
                r = jnp.maximum(r, 0.0)
            if scale_out:
                r = r * jnp.broadcast_to(dis_ref[:, :1], r.shape)
            out_ref[...] = r

    return pl.pallas_call(
        body,
        grid=grid,
        in_specs=[
            pl.BlockSpec((1, _BM, 128), lambda i, j, k: (k, i, 0)),
            pl.BlockSpec((_BM, 128), lambda i, j, k: (i, k)),
            pl.BlockSpec((_BM, 128), lambda i, j, k: (i, 0)),
            pl.BlockSpec((128, bn), lambda i, j, k: (k, j)),
            pl.BlockSpec((1, bn), lambda i, j, k: (0, j)),
        ],
        out_specs=pl.BlockSpec((_BM, bn), lambda i, j, k: (i, j)),
        out_shape=jax.ShapeDtypeStruct((N_PAD, dout), jnp.float32),
        compiler_params=pltpu.CompilerParams(
            dimension_semantics=("parallel", "parallel", "arbitrary")),
    )(S, g, dis, W, b)


def _mm_plain_scaled(A, dis, W, bm=_BM, bn=512, bk=512):
    din, dout = W.shape
    grid = (N_PAD // bm, dout // bn, din // bk)

    def body(a_ref, dis_ref, w_ref, out_ref):
        k = pl.program_id(2)
        nk = pl.num_programs(2)
        part = jnp.dot(a_ref[...], w_ref[...],
                       preferred_element_type=jnp.float32)

        @pl.when(k == 0)
        def _():
            out_ref[...] = part

        @pl.when(k > 0)
        def _():
            out_ref[...] += part

        @pl.when(k == nk - 1)
        def _():
            out_ref[...] = out_ref[...] * jnp.broadcast_to(
                dis_ref[:, :1], out_ref.shape)

    return pl.pallas_call(
        body,
        grid=grid,
        in_specs=[
            pl.BlockSpec((bm, bk), lambda i, j, k: (i, k)),
            pl.BlockSpec((bm, 128), lambda i, j, k: (i, 0)),
            pl.BlockSpec((bk, bn), lambda i, j, k: (k, j)),
        ],
        out_specs=pl.BlockSpec((bm, bn), lambda i, j, k: (i, j)),
        out_shape=jax.ShapeDtypeStruct((N_PAD, dout), jnp.float32),
        compiler_params=pltpu.CompilerParams(
            dimension_semantics=("parallel", "parallel", "arbitrary")),
    )(A, dis, W)


def _mm_final(S, g, dis, b4, Wo, bo):
    K = S.shape[0]
    grid = (N_PAD // _BM, K)

    def body(s_ref, g_ref, dis_ref, b4_ref, wo_ref, bo_ref, out_ref):
        k = pl.program_id(1)
        nk = pl.num_programs(1)
        h = (s_ref[0] + g_ref[...]) * dis_ref[...] + b4_ref[...]
        h = jnp.maximum(h, 0.0)
        part = jnp.dot(h, wo_ref[...], preferred_element_type=jnp.float32)

        @pl.when(k == 0)
        def _():
            out_ref[...] = part

        @pl.when(k > 0)
        def _():
            out_ref[...] += part

        @pl.when(k == nk - 1)
        def _():
            out_ref[...] += bo_ref[...]

    return pl.pallas_call(
        body,
        grid=grid,
        in_specs=[
            pl.BlockSpec((1, _BM, 128), lambda i, k: (k, i, 0)),
            pl.BlockSpec((_BM, 128), lambda i, k: (i, k)),
            pl.BlockSpec((_BM, 128), lambda i, k: (i, 0)),
            pl.BlockSpec((1, 128), lambda i, k: (0, k)),
            pl.BlockSpec((128, 128), lambda i, k: (k, 0)),
            pl.BlockSpec((1, 128), lambda i, k: (0, 0)),
        ],
        out_specs=pl.BlockSpec((_BM, 128), lambda i, k: (i, 0)),
        out_shape=jax.ShapeDtypeStruct((N_PAD, 128), jnp.float32),
        compiler_params=pltpu.CompilerParams(
            dimension_semantics=("parallel", "arbitrary")),
    )(S, g, dis, b4, Wo, bo)


_deg_kernel = _make_deg_kernel()
_agg_kernels = {K: _make_agg_kernel(K) for K in (2, 8, 16)}


def kernel(x, edge_index, W1, b1, W2, b2, W3, b3, W4, b4, Wo, bo):
    row = edge_index[0].astype(jnp.int32)
    col = edge_index[1].astype(jnp.int32)
    npad = E_PAD - E_EDGES
    pad_i = jnp.arange(npad, dtype=jnp.int32)
    rowp = jnp.concatenate([row, pad_i % N_NODES])
    colp = jnp.concatenate([col, N_NODES + pad_i % (SLAB_ROWS - N_NODES)])

    ones128 = jnp.ones((WIN, 128), jnp.float32)
    zeros128 = jnp.zeros((ROWS_PER_TILE, 128), jnp.float32)

    hist = _deg_kernel(colp, ones128, zeros128)
    x_p = jnp.pad(x, ((0, N_PAD - N_NODES), (0, 0)))
    dis, g0 = _prep(hist[0], hist[1], x_p)

    roww = rowp.reshape(NS, NWIN, WIN)
    colw = colp.reshape(NS, NWIN, WIN)

    def agg(g, K):
        gidxf = (roww[None] * K
                 + jnp.arange(K, dtype=jnp.int32)[:, None, None, None]
                 ).reshape(-1)
        return _agg_kernels[K](g.reshape(N_PAD * K, 128), gidxf, colp,
                               zeros128)

    S0 = agg(g0, 2)
    g1 = _mm_layer(S0, g0, dis, W1, b1.reshape(1, -1), relu=True,
                   scale_out=True)
    S1 = agg(g1, 8)
    g2 = _mm_layer(S1, g1, dis, W2, b2.reshape(1, -1), relu=True,
                   scale_out=True)
    S2 = agg(g2, 16)
    h3 = _mm_layer(S2, g2, dis, W3, b3.reshape(1, -1), relu=True,
                   scale_out=False)
    g3 = _mm_plain_scaled(h3, dis, W4)
    S3 = agg(g3, 16)
    out = _mm_final(S3, g3, dis, b4.reshape(1, -1), Wo, bo.reshape(1, -1))
    return out[:N_NODES]


# speedup vs baseline: 3.8792x; 1.7906x over previous
import functools

import jax
import jax.numpy as jnp
from jax import lax
from jax.experimental import pallas as pl
from jax.experimental.pallas import tpu as pltpu
from jax.experimental.pallas import tpu_sc as plsc

N_NODES = 10000
N_PAD = 10240
E_EDGES = 160000
E_PAD = 163840
NC, NS = 2, 16
WIN = 128
ROWS_PER_TILE = N_PAD // NS
EDGES_PER_W32 = E_PAD // (NC * NS)
EDGES_PER_W16 = E_PAD // NS

_MESH = dict(core_axis_name="c", subcore_axis_name="s", num_cores=NC,
             num_subcores=NS)


def _make_deg_kernel():
    mesh = plsc.VectorSubcoreMesh(**_MESH)

    @functools.partial(
        pl.kernel,
        out_type=jax.ShapeDtypeStruct((NC, N_PAD, 128), jnp.float32),
        mesh=mesh,
        scratch_types=[
            pltpu.VMEM_SHARED((N_PAD, 128), jnp.float32),
            pltpu.VMEM((WIN, 128), jnp.float32),
            pltpu.VMEM((WIN,), jnp.int32),
        ],
    )
    def deg_k(colp, ones_hbm, zeros_hbm, out, slab, ones_v, cidx):
        c = lax.axis_index("c")
        s = lax.axis_index("s")
        wid = c * NS + s
        pltpu.sync_copy(zeros_hbm, slab.at[pl.ds(s * ROWS_PER_TILE,
                                                 ROWS_PER_TILE)])
        pltpu.sync_copy(ones_hbm, ones_v)
        plsc.subcore_barrier()

        def body(w, carry):
            base = wid * EDGES_PER_W32 + w * WIN
            pltpu.sync_copy(colp.at[pl.ds(base, WIN)], cidx)
            pltpu.sync_copy(ones_v, slab.at[cidx], add=True)
            return carry

        lax.fori_loop(0, EDGES_PER_W32 // WIN, body, 0)
        plsc.subcore_barrier()
        pltpu.sync_copy(slab.at[pl.ds(s * ROWS_PER_TILE, ROWS_PER_TILE)],
                        out.at[c, pl.ds(s * ROWS_PER_TILE, ROWS_PER_TILE)])

    return deg_k


NWIN = EDGES_PER_W16 // WIN
NWIN_H = NWIN // 2
SLAB_ROWS = N_PAD
TROWS = SLAB_ROWS // NS


def _make_agg_kernel(K):
    mesh = plsc.VectorSubcoreMesh(**_MESH)
    cpc = K // NC

    @functools.partial(
        pl.kernel,
        out_type=jax.ShapeDtypeStruct((K, N_PAD, 128), jnp.float32),
        mesh=mesh,
        scratch_types=[
            pltpu.VMEM_SHARED((SLAB_ROWS, 128), jnp.float32),
            pltpu.VMEM((WIN, 128), jnp.float32),
            pltpu.VMEM((WIN, 128), jnp.float32),
            [pltpu.VMEM((WIN,), jnp.int32)] * 4,
            [pltpu.VMEM((WIN,), jnp.int32)] * 4,
            [pltpu.SemaphoreType.DMA] * 4,
            [pltpu.SemaphoreType.DMA] * 2,
            [pltpu.SemaphoreType.DMA] * 2,
        ],
    )
    def agg_k(g2df, gidxf, colf, zeros_hbm, out, slab, rows0, rows1,
              gbuf, cbuf, isem, gsem, ssem):
        cid = lax.axis_index("c")
        s = lax.axis_index("s")
        rows = (rows0, rows1)

        def idx_start(w, q, ch):
            gbase = (ch * NS + s) * EDGES_PER_W16 + w * WIN
            cbase = s * EDGES_PER_W16 + w * WIN
            pltpu.async_copy(gidxf.at[pl.ds(gbase, WIN)], gbuf[q],
                             isem[q])
            pltpu.async_copy(colf.at[pl.ds(cbase, WIN)], cbuf[q],
                             isem[q])

        def idx_wait(q):
            pltpu.make_async_copy(colf.at[pl.ds(0, WIN)], gbuf[q],
                                  isem[q]).wait()
            pltpu.make_async_copy(colf.at[pl.ds(0, WIN)], cbuf[q],
                                  isem[q]).wait()

        def gather_start(q, p):
            pltpu.async_copy(g2df.at[gbuf[q]], rows[p], gsem[p])

        def gather_wait(q, p):
            pltpu.make_async_copy(g2df.at[gbuf[q]], rows[p],
                                  gsem[p]).wait()

        def scatter_start(q, p):
            pltpu.async_copy(rows[p], slab.at[cbuf[q]], ssem[p],
                             add=True)

        def scatter_wait(q, p):
            pltpu.make_async_copy(rows[p], slab.at[cbuf[q]],
                                  ssem[p]).wait()

        def step(w, q, ch, do_idx, do_gather):
            p = q % 2
            gather_wait(q, p)
            scatter_start(q, p)
            scatter_wait(q, p)
            if do_idx:
                idx_start(w + 4, q, ch)
            if do_gather:
                q2 = (q + 2) % 4
                idx_wait(q2)
                gather_start(q2, p)

        for cl in range(cpc):
            ch = cid * cpc + cl
            pltpu.sync_copy(zeros_hbm, slab.at[pl.ds(s * TROWS, TROWS)])
            plsc.subcore_barrier()

            for w in range(4):
                idx_start(w, w, ch)
            for w in range(2):
                idx_wait(w)
                gather_start(w, w)

            def body(j, carry):
                for qq in range(4):
                    step(4 * j + qq, qq, ch, True, True)
                return carry

            lax.fori_loop(0, NWIN // 4 - 2, body, 0)
            for w in range(NWIN - 8, NWIN):
                step(w, w % 4, ch, w + 4 < NWIN, w + 2 < NWIN)
            plsc.subcore_barrier()
            pltpu.sync_copy(
                slab.at[pl.ds(s * TROWS, TROWS)],
                out.at[ch, pl.ds(s * TROWS, TROWS)])
            plsc.subcore_barrier()

    return agg_k


_BM = 256


def _prep_body(h0_ref, h1_ref, x_ref, dis_ref, g0_ref):
    deg128 = h0_ref[...] + h1_ref[...] + 1.0
    dis_ref[...] = lax.rsqrt(deg128)
    dis = lax.rsqrt(deg128[:, :1])
    g0_ref[...] = x_ref[...] * dis


def _prep(h0, h1, x_p):
    grid = (N_PAD // _BM,)
    return pl.pallas_call(
        _prep_body,
        grid=grid,
        in_specs=[
            pl.BlockSpec((_BM, 128), lambda i: (i, 0)),
            pl.BlockSpec((_BM, 128), lambda i: (i, 0)),
            pl.BlockSpec((_BM, 256), lambda i: (i, 0)),
        ],
        out_specs=[
            pl.BlockSpec((_BM, 128), lambda i: (i, 0)),
            pl.BlockSpec((_BM, 256), lambda i: (i, 0)),
        ],
        out_shape=[
            jax.ShapeDtypeStruct((N_PAD, 128), jnp.float32),
            jax.ShapeDtypeStruct((N_PAD, 256), jnp.float32),
        ],
    )(h0, h1, x_p)


def _mm_layer(S, g, dis, W, b, relu, scale_out, bm=512, bn=1024):
    K = S.shape[0]
    dout = W.shape[1]
    bn = min(bn, dout)
    grid = (N_PAD // bm, dout // bn, K)

    def body(s_ref, g_ref, dis_ref, w_ref, b_ref, out_ref):
        k = pl.program_id(2)
        nk = pl.num_programs(2)
        a = (s_ref[0] + g_ref[...]) * dis_ref[...]
        part = jnp.dot(a, w_ref[...], preferred_element_type=jnp.float32)

        @pl.when(k == 0)
        def _():
            out_ref[...] = part

        @pl.when(k > 0)
        def _():
            out_ref[...] += part

        @pl.when(k == nk - 1)
        def _():
            r = out_ref[...] + b_ref[...]
            if relu:
                r = jnp.maximum(r, 0.0)
            if scale_out:
                r = r * jnp.broadcast_to(dis_ref[:, :1], r.shape)
            out_ref[...] = r

    return pl.pallas_call(
        body,
        grid=grid,
        in_specs=[
            pl.BlockSpec((1, bm, 128), lambda i, j, k: (k, i, 0)),
            pl.BlockSpec((bm, 128), lambda i, j, k: (i, k)),
            pl.BlockSpec((bm, 128), lambda i, j, k: (i, 0)),
            pl.BlockSpec((128, bn), lambda i, j, k: (k, j)),
            pl.BlockSpec((1, bn), lambda i, j, k: (0, j)),
        ],
        out_specs=pl.BlockSpec((bm, bn), lambda i, j, k: (i, j)),
        out_shape=jax.ShapeDtypeStruct((N_PAD, dout), jnp.float32),
        compiler_params=pltpu.CompilerParams(
            dimension_semantics=("parallel", "parallel", "arbitrary")),
    )(S, g, dis, W, b)


def _mm_plain_scaled(A, dis, W, bm=512, bn=1024, bk=512):
    din, dout = W.shape
    grid = (N_PAD // bm, dout // bn, din // bk)

    def body(a_ref, dis_ref, w_ref, out_ref):
        k = pl.program_id(2)
        nk = pl.num_programs(2)
        part = jnp.dot(a_ref[...], w_ref[...],
                       preferred_element_type=jnp.float32)

        @pl.when(k == 0)
        def _():
            out_ref[...] = part

        @pl.when(k > 0)
        def _():
            out_ref[...] += part

        @pl.when(k == nk - 1)
        def _():
            out_ref[...] = out_ref[...] * jnp.broadcast_to(
                dis_ref[:, :1], out_ref.shape)

    return pl.pallas_call(
        body,
        grid=grid,
        in_specs=[
            pl.BlockSpec((bm, bk), lambda i, j, k: (i, k)),
            pl.BlockSpec((bm, 128), lambda i, j, k: (i, 0)),
            pl.BlockSpec((bk, bn), lambda i, j, k: (k, j)),
        ],
        out_specs=pl.BlockSpec((bm, bn), lambda i, j, k: (i, j)),
        out_shape=jax.ShapeDtypeStruct((N_PAD, dout), jnp.float32),
        compiler_params=pltpu.CompilerParams(
            dimension_semantics=("parallel", "parallel", "arbitrary")),
    )(A, dis, W)


def _mm_final(S, g, dis, b4, Wo, bo, bm=512):
    K = S.shape[0]
    grid = (N_PAD // bm, K)

    def body(s_ref, g_ref, dis_ref, b4_ref, wo_ref, bo_ref, out_ref):
        k = pl.program_id(1)
        nk = pl.num_programs(1)
        h = (s_ref[0] + g_ref[...]) * dis_ref[...] + b4_ref[...]
        h = jnp.maximum(h, 0.0)
        part = jnp.dot(h, wo_ref[...], preferred_element_type=jnp.float32)

        @pl.when(k == 0)
        def _():
            out_ref[...] = part

        @pl.when(k > 0)
        def _():
            out_ref[...] += part

        @pl.when(k == nk - 1)
        def _():
            out_ref[...] += bo_ref[...]

    return pl.pallas_call(
        body,
        grid=grid,
        in_specs=[
            pl.BlockSpec((1, bm, 128), lambda i, k: (k, i, 0)),
            pl.BlockSpec((bm, 128), lambda i, k: (i, k)),
            pl.BlockSpec((bm, 128), lambda i, k: (i, 0)),
            pl.BlockSpec((1, 128), lambda i, k: (0, k)),
            pl.BlockSpec((128, 128), lambda i, k: (k, 0)),
            pl.BlockSpec((1, 128), lambda i, k: (0, 0)),
        ],
        out_specs=pl.BlockSpec((bm, 128), lambda i, k: (i, 0)),
        out_shape=jax.ShapeDtypeStruct((N_PAD, 128), jnp.float32),
        compiler_params=pltpu.CompilerParams(
            dimension_semantics=("parallel", "arbitrary")),
    )(S, g, dis, b4, Wo, bo)


_deg_kernel = _make_deg_kernel()
_agg_kernels = {K: _make_agg_kernel(K) for K in (2, 8, 16)}


def kernel(x, edge_index, W1, b1, W2, b2, W3, b3, W4, b4, Wo, bo):
    row = edge_index[0].astype(jnp.int32)
    col = edge_index[1].astype(jnp.int32)
    npad = E_PAD - E_EDGES
    pad_i = jnp.arange(npad, dtype=jnp.int32)
    rowp = jnp.concatenate([row, pad_i % N_NODES])
    colp = jnp.concatenate([col, N_NODES + pad_i % (SLAB_ROWS - N_NODES)])

    ones128 = jnp.ones((WIN, 128), jnp.float32)
    zeros128 = jnp.zeros((ROWS_PER_TILE, 128), jnp.float32)

    hist = _deg_kernel(colp, ones128, zeros128)
    x_p = jnp.pad(x, ((0, N_PAD - N_NODES), (0, 0)))
    dis, g0 = _prep(hist[0], hist[1], x_p)

    roww = rowp.reshape(NS, NWIN, WIN)
    colw = colp.reshape(NS, NWIN, WIN)

    def agg(g, K):
        gidxf = (roww[None] * K
                 + jnp.arange(K, dtype=jnp.int32)[:, None, None, None]
                 ).reshape(-1)
        return _agg_kernels[K](g.reshape(N_PAD * K, 128), gidxf, colp,
                               zeros128)

    S0 = agg(g0, 2)
    g1 = _mm_layer(S0, g0, dis, W1, b1.reshape(1, -1), relu=True,
                   scale_out=True)
    S1 = agg(g1, 8)
    g2 = _mm_layer(S1, g1, dis, W2, b2.reshape(1, -1), relu=True,
                   scale_out=True)
    S2 = agg(g2, 16)
    h3 = _mm_layer(S2, g2, dis, W3, b3.reshape(1, -1), relu=True,
                   scale_out=False)
    g3 = _mm_plain_scaled(h3, dis, W4)
    S3 = agg(g3, 16)
    out = _mm_final(S3, g3, dis, b4.reshape(1, -1), Wo, bo.reshape(1, -1))
    return out[:N_NODES]


# speedup vs baseline: 3.9612x; 1.0211x over previous
import functools

import jax
import jax.numpy as jnp
from jax import lax
from jax.experimental import pallas as pl
from jax.experimental.pallas import tpu as pltpu
from jax.experimental.pallas import tpu_sc as plsc

N_NODES = 10000
N_PAD = 10240
E_EDGES = 160000
E_PAD = 163840
NC, NS = 2, 16
WIN = 128
ROWS_PER_TILE = N_PAD // NS
EDGES_PER_W32 = E_PAD // (NC * NS)
EDGES_PER_W16 = E_PAD // NS

_MESH = dict(core_axis_name="c", subcore_axis_name="s", num_cores=NC,
             num_subcores=NS)


def _make_deg_kernel():
    mesh = plsc.VectorSubcoreMesh(**_MESH)

    @functools.partial(
        pl.kernel,
        out_type=jax.ShapeDtypeStruct((NC, N_PAD, 128), jnp.float32),
        mesh=mesh,
        scratch_types=[
            pltpu.VMEM_SHARED((N_PAD, 128), jnp.float32),
            pltpu.VMEM((WIN, 128), jnp.float32),
            pltpu.VMEM((WIN,), jnp.int32),
        ],
    )
    def deg_k(colp, ones_hbm, zeros_hbm, out, slab, ones_v, cidx):
        c = lax.axis_index("c")
        s = lax.axis_index("s")
        wid = c * NS + s
        pltpu.sync_copy(zeros_hbm, slab.at[pl.ds(s * ROWS_PER_TILE,
                                                 ROWS_PER_TILE)])
        pltpu.sync_copy(ones_hbm, ones_v)
        plsc.subcore_barrier()

        def body(w, carry):
            base = wid * EDGES_PER_W32 + w * WIN
            pltpu.sync_copy(colp.at[pl.ds(base, WIN)], cidx)
            pltpu.sync_copy(ones_v, slab.at[cidx], add=True)
            return carry

        lax.fori_loop(0, EDGES_PER_W32 // WIN, body, 0)
        plsc.subcore_barrier()
        pltpu.sync_copy(slab.at[pl.ds(s * ROWS_PER_TILE, ROWS_PER_TILE)],
                        out.at[c, pl.ds(s * ROWS_PER_TILE, ROWS_PER_TILE)])

    return deg_k


NWIN = EDGES_PER_W16 // WIN
NWIN_H = NWIN // 2
SLAB_ROWS = N_PAD
TROWS = SLAB_ROWS // NS


def _make_agg_kernel(K):
    mesh = plsc.VectorSubcoreMesh(**_MESH)
    cpc = K // NC

    @functools.partial(
        pl.kernel,
        out_type=jax.ShapeDtypeStruct((K, N_PAD, 128), jnp.float32),
        mesh=mesh,
        scratch_types=[
            pltpu.VMEM_SHARED((SLAB_ROWS, 128), jnp.float32),
            pltpu.VMEM((WIN, 128), jnp.float32),
            pltpu.VMEM((WIN, 128), jnp.float32),
            [pltpu.VMEM((WIN,), jnp.int32)] * 4,
            [pltpu.VMEM((WIN,), jnp.int32)] * 4,
            [pltpu.SemaphoreType.DMA] * 4,
            [pltpu.SemaphoreType.DMA] * 2,
            [pltpu.SemaphoreType.DMA] * 2,
        ],
    )
    def agg_k(g2df, gidxf, colf, zeros_hbm, out, slab, rows0, rows1,
              gbuf, cbuf, isem, gsem, ssem):
        cid = lax.axis_index("c")
        s = lax.axis_index("s")
        rows = (rows0, rows1)

        def idx_start(w, q, ch):
            gbase = (ch * NS + s) * EDGES_PER_W16 + w * WIN
            cbase = s * EDGES_PER_W16 + w * WIN
            pltpu.async_copy(gidxf.at[pl.ds(gbase, WIN)], gbuf[q],
                             isem[q])
            pltpu.async_copy(colf.at[pl.ds(cbase, WIN)], cbuf[q],
                             isem[q])

        def idx_wait(q):
            pltpu.make_async_copy(colf.at[pl.ds(0, WIN)], gbuf[q],
                                  isem[q]).wait()
            pltpu.make_async_copy(colf.at[pl.ds(0, WIN)], cbuf[q],
                                  isem[q]).wait()

        def gather_start(q, p):
            pltpu.async_copy(g2df.at[gbuf[q]], rows[p], gsem[p])

        def gather_wait(q, p):
            pltpu.make_async_copy(g2df.at[gbuf[q]], rows[p],
                                  gsem[p]).wait()

        def scatter_start(q, p):
            pltpu.async_copy(rows[p], slab.at[cbuf[q]], ssem[p],
                             add=True)

        def scatter_wait(q, p):
            pltpu.make_async_copy(rows[p], slab.at[cbuf[q]],
                                  ssem[p]).wait()

        def step(w, q, ch, do_idx, do_gather):
            p = q % 2
            gather_wait(q, p)
            scatter_start(q, p)
            scatter_wait(q, p)
            if do_idx:
                idx_start(w + 4, q, ch)
            if do_gather:
                q2 = (q + 2) % 4
                idx_wait(q2)
                gather_start(q2, p)

        for cl in range(cpc):
            ch = cid * cpc + cl
            pltpu.sync_copy(zeros_hbm, slab.at[pl.ds(s * TROWS, TROWS)])
            plsc.subcore_barrier()

            for w in range(4):
                idx_start(w, w, ch)
            for w in range(2):
                idx_wait(w)
                gather_start(w, w)

            def body(j, carry):
                for qq in range(4):
                    step(4 * j + qq, qq, ch, True, True)
                return carry

            lax.fori_loop(0, NWIN // 4 - 2, body, 0)
            for w in range(NWIN - 8, NWIN):
                step(w, w % 4, ch, w + 4 < NWIN, w + 2 < NWIN)
            plsc.subcore_barrier()
            pltpu.sync_copy(
                slab.at[pl.ds(s * TROWS, TROWS)],
                out.at[ch, pl.ds(s * TROWS, TROWS)])
            plsc.subcore_barrier()

    return agg_k


_BM = 256


def _prep_body(h0_ref, h1_ref, x_ref, dis_ref, g0_ref):
    deg128 = h0_ref[...] + h1_ref[...] + 1.0
    dis_ref[...] = lax.rsqrt(deg128)
    dis = lax.rsqrt(deg128[:, :1])
    g0_ref[...] = x_ref[...] * dis


def _prep(h0, h1, x_p):
    grid = (N_PAD // _BM,)
    return pl.pallas_call(
        _prep_body,
        grid=grid,
        in_specs=[
            pl.BlockSpec((_BM, 128), lambda i: (i, 0)),
            pl.BlockSpec((_BM, 128), lambda i: (i, 0)),
            pl.BlockSpec((_BM, 256), lambda i: (i, 0)),
        ],
        out_specs=[
            pl.BlockSpec((_BM, 128), lambda i: (i, 0)),
            pl.BlockSpec((_BM, 256), lambda i: (i, 0)),
        ],
        out_shape=[
            jax.ShapeDtypeStruct((N_PAD, 128), jnp.float32),
            jax.ShapeDtypeStruct((N_PAD, 256), jnp.float32),
        ],
    )(h0, h1, x_p)


def _mm_layer(S, g, dis, W, b, relu, scale_out, bm=512, bn=1024):
    K = S.shape[0]
    dout = W.shape[1]
    bn = min(bn, dout)
    grid = (N_PAD // bm, dout // bn, K)

    def body(s_ref, g_ref, dis_ref, w_ref, b_ref, out_ref):
        k = pl.program_id(2)
        nk = pl.num_programs(2)
        a = (s_ref[0] + g_ref[...]) * dis_ref[...]
        part = jnp.dot(a.astype(jnp.bfloat16), w_ref[...],
                       preferred_element_type=jnp.float32)

        @pl.when(k == 0)
        def _():
            out_ref[...] = part

        @pl.when(k > 0)
        def _():
            out_ref[...] += part

        @pl.when(k == nk - 1)
        def _():
            r = out_ref[...] + b_ref[...]
            if relu:
                r = jnp.maximum(r, 0.0)
            if scale_out:
                r = r * jnp.broadcast_to(dis_ref[:, :1], r.shape)
            out_ref[...] = r

    return pl.pallas_call(
        body,
        grid=grid,
        in_specs=[
            pl.BlockSpec((1, bm, 128), lambda i, j, k: (k, i, 0)),
            pl.BlockSpec((bm, 128), lambda i, j, k: (i, k)),
            pl.BlockSpec((bm, 128), lambda i, j, k: (i, 0)),
            pl.BlockSpec((128, bn), lambda i, j, k: (k, j)),
            pl.BlockSpec((1, bn), lambda i, j, k: (0, j)),
        ],
        out_specs=pl.BlockSpec((bm, bn), lambda i, j, k: (i, j)),
        out_shape=jax.ShapeDtypeStruct((N_PAD, dout), jnp.float32),
        compiler_params=pltpu.CompilerParams(
            dimension_semantics=("parallel", "parallel", "arbitrary")),
    )(S, g, dis, W, b)


def _mm_plain_scaled(A, dis, W, bm=512, bn=1024, bk=512):
    din, dout = W.shape
    grid = (N_PAD // bm, dout // bn, din // bk)

    def body(a_ref, dis_ref, w_ref, out_ref):
        k = pl.program_id(2)
        nk = pl.num_programs(2)
        part = jnp.dot(a_ref[...].astype(jnp.bfloat16), w_ref[...],
                       preferred_element_type=jnp.float32)

        @pl.when(k == 0)
        def _():
            out_ref[...] = part

        @pl.when(k > 0)
        def _():
            out_ref[...] += part

        @pl.when(k == nk - 1)
        def _():
            out_ref[...] = out_ref[...] * jnp.broadcast_to(
                dis_ref[:, :1], out_ref.shape)

    return pl.pallas_call(
        body,
        grid=grid,
        in_specs=[
            pl.BlockSpec((bm, bk), lambda i, j, k: (i, k)),
            pl.BlockSpec((bm, 128), lambda i, j, k: (i, 0)),
            pl.BlockSpec((bk, bn), lambda i, j, k: (k, j)),
        ],
        out_specs=pl.BlockSpec((bm, bn), lambda i, j, k: (i, j)),
        out_shape=jax.ShapeDtypeStruct((N_PAD, dout), jnp.float32),
        compiler_params=pltpu.CompilerParams(
            dimension_semantics=("parallel", "parallel", "arbitrary")),
    )(A, dis, W)


def _mm_final(S, g, dis, b4, Wo, bo, bm=512):
    K = S.shape[0]
    grid = (N_PAD // bm, K)

    def body(s_ref, g_ref, dis_ref, b4_ref, wo_ref, bo_ref, out_ref):
        k = pl.program_id(1)
        nk = pl.num_programs(1)
        h = (s_ref[0] + g_ref[...]) * dis_ref[...] + b4_ref[...]
        h = jnp.maximum(h, 0.0)
        part = jnp.dot(h.astype(jnp.bfloat16), wo_ref[...],
                       preferred_element_type=jnp.float32)

        @pl.when(k == 0)
        def _():
            out_ref[...] = part

        @pl.when(k > 0)
        def _():
            out_ref[...] += part

        @pl.when(k == nk - 1)
        def _():
            out_ref[...] += bo_ref[...]

    return pl.pallas_call(
        body,
        grid=grid,
        in_specs=[
            pl.BlockSpec((1, bm, 128), lambda i, k: (k, i, 0)),
            pl.BlockSpec((bm, 128), lambda i, k: (i, k)),
            pl.BlockSpec((bm, 128), lambda i, k: (i, 0)),
            pl.BlockSpec((1, 128), lambda i, k: (0, k)),
            pl.BlockSpec((128, 128), lambda i, k: (k, 0)),
            pl.BlockSpec((1, 128), lambda i, k: (0, 0)),
        ],
        out_specs=pl.BlockSpec((bm, 128), lambda i, k: (i, 0)),
        out_shape=jax.ShapeDtypeStruct((N_PAD, 128), jnp.float32),
        compiler_params=pltpu.CompilerParams(
            dimension_semantics=("parallel", "arbitrary")),
    )(S, g, dis, b4, Wo, bo)


_deg_kernel = _make_deg_kernel()
_agg_kernels = {K: _make_agg_kernel(K) for K in (2, 8, 16)}


def kernel(x, edge_index, W1, b1, W2, b2, W3, b3, W4, b4, Wo, bo):
    row = edge_index[0].astype(jnp.int32)
    col = edge_index[1].astype(jnp.int32)
    npad = E_PAD - E_EDGES
    pad_i = jnp.arange(npad, dtype=jnp.int32)
    rowp = jnp.concatenate([row, pad_i % N_NODES])
    colp = jnp.concatenate([col, N_NODES + pad_i % (SLAB_ROWS - N_NODES)])

    ones128 = jnp.ones((WIN, 128), jnp.float32)
    zeros128 = jnp.zeros((ROWS_PER_TILE, 128), jnp.float32)

    hist = _deg_kernel(colp, ones128, zeros128)
    x_p = jnp.pad(x, ((0, N_PAD - N_NODES), (0, 0)))
    dis, g0 = _prep(hist[0], hist[1], x_p)

    roww = rowp.reshape(NS, NWIN, WIN)
    colw = colp.reshape(NS, NWIN, WIN)

    def agg(g, K):
        gidxf = (roww[None] * K
                 + jnp.arange(K, dtype=jnp.int32)[:, None, None, None]
                 ).reshape(-1)
        return _agg_kernels[K](g.reshape(N_PAD * K, 128), gidxf, colp,
                               zeros128)

    bf = jnp.bfloat16
    S0 = agg(g0, 2)
    g1 = _mm_layer(S0, g0, dis, W1.astype(bf), b1.reshape(1, -1),
                   relu=True, scale_out=True)
    S1 = agg(g1, 8)
    g2 = _mm_layer(S1, g1, dis, W2.astype(bf), b2.reshape(1, -1),
                   relu=True, scale_out=True)
    S2 = agg(g2, 16)
    h3 = _mm_layer(S2, g2, dis, W3.astype(bf), b3.reshape(1, -1),
                   relu=True, scale_out=False)
    g3 = _mm_plain_scaled(h3, dis, W4.astype(bf))
    S3 = agg(g3, 16)
    out = _mm_final(S3, g3, dis, b4.reshape(1, -1), Wo.astype(bf),
                    bo.reshape(1, -1))
    return out[:N_NODES]


# speedup vs baseline: 3.9621x; 1.0002x over previous
import functools

import jax
import jax.numpy as jnp
from jax import lax
from jax.experimental import pallas as pl
from jax.experimental.pallas import tpu as pltpu
from jax.experimental.pallas import tpu_sc as plsc

N_NODES = 10000
N_PAD = 10240
E_EDGES = 160000
E_PAD = 163840
NC, NS = 2, 16
WIN = 128
ROWS_PER_TILE = N_PAD // NS
EDGES_PER_W32 = E_PAD // (NC * NS)
EDGES_PER_W16 = E_PAD // NS

_MESH = dict(core_axis_name="c", subcore_axis_name="s", num_cores=NC,
             num_subcores=NS)


def _make_deg_kernel():
    mesh = plsc.VectorSubcoreMesh(**_MESH)

    @functools.partial(
        pl.kernel,
        out_type=jax.ShapeDtypeStruct((NC, N_PAD, 128), jnp.float32),
        mesh=mesh,
        scratch_types=[
            pltpu.VMEM_SHARED((N_PAD, 128), jnp.float32),
            pltpu.VMEM((WIN, 128), jnp.float32),
            pltpu.VMEM((WIN,), jnp.int32),
        ],
    )
    def deg_k(colp, ones_hbm, zeros_hbm, out, slab, ones_v, cidx):
        c = lax.axis_index("c")
        s = lax.axis_index("s")
        wid = c * NS + s
        pltpu.sync_copy(zeros_hbm, slab.at[pl.ds(s * ROWS_PER_TILE,
                                                 ROWS_PER_TILE)])
        pltpu.sync_copy(ones_hbm, ones_v)
        plsc.subcore_barrier()

        def body(w, carry):
            base = wid * EDGES_PER_W32 + w * WIN
            pltpu.sync_copy(colp.at[pl.ds(base, WIN)], cidx)
            pltpu.sync_copy(ones_v, slab.at[cidx], add=True)
            return carry

        lax.fori_loop(0, EDGES_PER_W32 // WIN, body, 0)
        plsc.subcore_barrier()
        pltpu.sync_copy(slab.at[pl.ds(s * ROWS_PER_TILE, ROWS_PER_TILE)],
                        out.at[c, pl.ds(s * ROWS_PER_TILE, ROWS_PER_TILE)])

    return deg_k


NWIN = EDGES_PER_W16 // WIN
NWIN_H = NWIN // 2
SLAB_ROWS = N_PAD
TROWS = SLAB_ROWS // NS


def _make_agg_kernel(K):
    mesh = plsc.VectorSubcoreMesh(**_MESH)
    cpc = K // NC

    @functools.partial(
        pl.kernel,
        out_type=jax.ShapeDtypeStruct((K, N_PAD, 128), jnp.float32),
        mesh=mesh,
        scratch_types=[
            pltpu.VMEM_SHARED((SLAB_ROWS, 128), jnp.float32),
            pltpu.VMEM((WIN, 128), jnp.float32),
            pltpu.VMEM((WIN, 128), jnp.float32),
            [pltpu.VMEM((WIN,), jnp.int32)] * 4,
            [pltpu.VMEM((WIN,), jnp.int32)] * 4,
            [pltpu.SemaphoreType.DMA] * 4,
            [pltpu.SemaphoreType.DMA] * 2,
            [pltpu.SemaphoreType.DMA] * 2,
        ],
    )
    def agg_k(g2df, gidxf, colf, zeros_hbm, out, slab, rows0, rows1,
              gbuf, cbuf, isem, gsem, ssem):
        cid = lax.axis_index("c")
        s = lax.axis_index("s")
        rows = (rows0, rows1)

        def idx_start(w, q, ch):
            gbase = (ch * NS + s) * EDGES_PER_W16 + w * WIN
            cbase = s * EDGES_PER_W16 + w * WIN
            pltpu.async_copy(gidxf.at[pl.ds(gbase, WIN)], gbuf[q],
                             isem[q])
            pltpu.async_copy(colf.at[pl.ds(cbase, WIN)], cbuf[q],
                             isem[q])

        def idx_wait(q):
            pltpu.make_async_copy(colf.at[pl.ds(0, WIN)], gbuf[q],
                                  isem[q]).wait()
            pltpu.make_async_copy(colf.at[pl.ds(0, WIN)], cbuf[q],
                                  isem[q]).wait()

        def gather_start(q, p):
            pltpu.async_copy(g2df.at[gbuf[q]], rows[p], gsem[p])

        def gather_wait(q, p):
            pltpu.make_async_copy(g2df.at[gbuf[q]], rows[p],
                                  gsem[p]).wait()

        def scatter_start(q, p):
            pltpu.async_copy(rows[p], slab.at[cbuf[q]], ssem[p],
                             add=True)

        def scatter_wait(q, p):
            pltpu.make_async_copy(rows[p], slab.at[cbuf[q]],
                                  ssem[p]).wait()

        def step(w, q, ch, do_idx, do_gather):
            p = q % 2
            gather_wait(q, p)
            scatter_start(q, p)
            scatter_wait(q, p)
            if do_idx:
                idx_start(w + 4, q, ch)
            if do_gather:
                q2 = (q + 2) % 4
                idx_wait(q2)
                gather_start(q2, p)

        for cl in range(cpc):
            ch = cid * cpc + cl
            pltpu.sync_copy(zeros_hbm, slab.at[pl.ds(s * TROWS, TROWS)])
            plsc.subcore_barrier()

            for w in range(4):
                idx_start(w, w, ch)
            for w in range(2):
                idx_wait(w)
                gather_start(w, w)

            def body(j, carry):
                for qq in range(4):
                    step(4 * j + qq, qq, ch, True, True)
                return carry

            lax.fori_loop(0, NWIN // 4 - 2, body, 0)
            for w in range(NWIN - 8, NWIN):
                step(w, w % 4, ch, w + 4 < NWIN, w + 2 < NWIN)
            plsc.subcore_barrier()
            pltpu.sync_copy(
                slab.at[pl.ds(s * TROWS, TROWS)],
                out.at[ch, pl.ds(s * TROWS, TROWS)])
            plsc.subcore_barrier()

    return agg_k


_BM = 256


def _prep_body(h0_ref, h1_ref, x_ref, dis_ref, g0_ref):
    deg128 = h0_ref[...] + h1_ref[...] + 1.0
    dis_ref[...] = lax.rsqrt(deg128)
    dis = lax.rsqrt(deg128[:, :1])
    g0_ref[...] = x_ref[...] * dis


def _prep(h0, h1, x_p):
    grid = (N_PAD // _BM,)
    return pl.pallas_call(
        _prep_body,
        grid=grid,
        in_specs=[
            pl.BlockSpec((_BM, 128), lambda i: (i, 0)),
            pl.BlockSpec((_BM, 128), lambda i: (i, 0)),
            pl.BlockSpec((_BM, 256), lambda i: (i, 0)),
        ],
        out_specs=[
            pl.BlockSpec((_BM, 128), lambda i: (i, 0)),
            pl.BlockSpec((_BM, 256), lambda i: (i, 0)),
        ],
        out_shape=[
            jax.ShapeDtypeStruct((N_PAD, 128), jnp.float32),
            jax.ShapeDtypeStruct((N_PAD, 256), jnp.float32),
        ],
    )(h0, h1, x_p)


def _mm_layer(S, g, dis, W, b, relu, scale_out, bm=512, bn=1024):
    K = S.shape[0]
    dout = W.shape[1]
    bn = min(bn, dout)
    grid = (N_PAD // bm, dout // bn, K)

    def body(s_ref, g_ref, dis_ref, w_ref, b_ref, out_ref):
        k = pl.program_id(2)
        nk = pl.num_programs(2)
        a = (s_ref[0] + g_ref[...]) * dis_ref[...]
        part = jnp.dot(a.astype(jnp.bfloat16), w_ref[...],
                       preferred_element_type=jnp.float32)

        @pl.when(k == 0)
        def _():
            out_ref[...] = part

        @pl.when(k > 0)
        def _():
            out_ref[...] += part

        @pl.when(k == nk - 1)
        def _():
            r = out_ref[...] + b_ref[...]
            if relu:
                r = jnp.maximum(r, 0.0)
            if scale_out:
                r = r * jnp.broadcast_to(dis_ref[:, :1], r.shape)
            out_ref[...] = r

    return pl.pallas_call(
        body,
        grid=grid,
        in_specs=[
            pl.BlockSpec((1, bm, 128), lambda i, j, k: (k, i, 0)),
            pl.BlockSpec((bm, 128), lambda i, j, k: (i, k)),
            pl.BlockSpec((bm, 128), lambda i, j, k: (i, 0)),
            pl.BlockSpec((128, bn), lambda i, j, k: (k, j)),
            pl.BlockSpec((1, bn), lambda i, j, k: (0, j)),
        ],
        out_specs=pl.BlockSpec((bm, bn), lambda i, j, k: (i, j)),
        out_shape=jax.ShapeDtypeStruct((N_PAD, dout), jnp.float32),
        compiler_params=pltpu.CompilerParams(
            dimension_semantics=("parallel", "parallel", "arbitrary")),
    )(S, g, dis, W, b)


def _mm_plain_scaled(A, dis, W, bm=512, bn=1024, bk=512):
    din, dout = W.shape
    grid = (N_PAD // bm, dout // bn, din // bk)

    def body(a_ref, dis_ref, w_ref, out_ref):
        k = pl.program_id(2)
        nk = pl.num_programs(2)
        part = jnp.dot(a_ref[...].astype(jnp.bfloat16), w_ref[...],
                       preferred_element_type=jnp.float32)

        @pl.when(k == 0)
        def _():
            out_ref[...] = part

        @pl.when(k > 0)
        def _():
            out_ref[...] += part

        @pl.when(k == nk - 1)
        def _():
            out_ref[...] = out_ref[...] * jnp.broadcast_to(
                dis_ref[:, :1], out_ref.shape)

    return pl.pallas_call(
        body,
        grid=grid,
        in_specs=[
            pl.BlockSpec((bm, bk), lambda i, j, k: (i, k)),
            pl.BlockSpec((bm, 128), lambda i, j, k: (i, 0)),
            pl.BlockSpec((bk, bn), lambda i, j, k: (k, j)),
        ],
        out_specs=pl.BlockSpec((bm, bn), lambda i, j, k: (i, j)),
        out_shape=jax.ShapeDtypeStruct((N_PAD, dout), jnp.float32),
        compiler_params=pltpu.CompilerParams(
            dimension_semantics=("parallel", "parallel", "arbitrary")),
    )(A, dis, W)


def _mm_final(S, g, dis, b4, Wo, bo, bm=512):
    K = S.shape[0]
    grid = (N_PAD // bm, K)

    def body(s_ref, g_ref, dis_ref, b4_ref, wo_ref, bo_ref, out_ref):
        k = pl.program_id(1)
        nk = pl.num_programs(1)
        h = (s_ref[0] + g_ref[...]) * dis_ref[...] + b4_ref[...]
        h = jnp.maximum(h, 0.0)
        part = jnp.dot(h.astype(jnp.bfloat16), wo_ref[...],
                       preferred_element_type=jnp.float32)

        @pl.when(k == 0)
        def _():
            out_ref[...] = part

        @pl.when(k > 0)
        def _():
            out_ref[...] += part

        @pl.when(k == nk - 1)
        def _():
            out_ref[...] += bo_ref[...]

    return pl.pallas_call(
        body,
        grid=grid,
        in_specs=[
            pl.BlockSpec((1, bm, 128), lambda i, k: (k, i, 0)),
            pl.BlockSpec((bm, 128), lambda i, k: (i, k)),
            pl.BlockSpec((bm, 128), lambda i, k: (i, 0)),
            pl.BlockSpec((1, 128), lambda i, k: (0, k)),
            pl.BlockSpec((128, 128), lambda i, k: (k, 0)),
            pl.BlockSpec((1, 128), lambda i, k: (0, 0)),
        ],
        out_specs=pl.BlockSpec((bm, 128), lambda i, k: (i, 0)),
        out_shape=jax.ShapeDtypeStruct((N_PAD, 128), jnp.float32),
        compiler_params=pltpu.CompilerParams(
            dimension_semantics=("parallel", "arbitrary")),
    )(S, g, dis, b4, Wo, bo)


_deg_kernel = _make_deg_kernel()
_agg_kernels = {K: _make_agg_kernel(K) for K in (2, 8, 16)}


def kernel(x, edge_index, W1, b1, W2, b2, W3, b3, W4, b4, Wo, bo):
    row = edge_index[0].astype(jnp.int32)
    col = edge_index[1].astype(jnp.int32)
    npad = E_PAD - E_EDGES
    pad_i = jnp.arange(npad, dtype=jnp.int32)
    rowp = jnp.concatenate([row, pad_i % N_NODES])
    colp = jnp.concatenate([col, N_NODES + pad_i % (SLAB_ROWS - N_NODES)])

    ones128 = jnp.ones((WIN, 128), jnp.float32)
    zeros128 = jnp.zeros((ROWS_PER_TILE, 128), jnp.float32)

    hist = _deg_kernel(colp, ones128, zeros128)
    x_p = jnp.pad(x, ((0, N_PAD - N_NODES), (0, 0)))
    dis, g0 = _prep(hist[0], hist[1], x_p)

    roww = rowp.reshape(NS, NWIN, WIN)

    def agg(g, K):
        gidxf = (roww[None] * K
                 + jnp.arange(K, dtype=jnp.int32)[:, None, None, None]
                 ).reshape(-1)
        return _agg_kernels[K](g.reshape(N_PAD * K, 128), gidxf, colp,
                               zeros128)

    bf = jnp.bfloat16
    S0 = agg(g0, 2)
    g1 = _mm_layer(S0, g0, dis, W1.astype(bf), b1.reshape(1, -1),
                   relu=True, scale_out=True)
    S1 = agg(g1, 8)
    g2 = _mm_layer(S1, g1, dis, W2.astype(bf), b2.reshape(1, -1),
                   relu=True, scale_out=True)
    S2 = agg(g2, 16)
    h3 = _mm_layer(S2, g2, dis, W3.astype(bf), b3.reshape(1, -1),
                   relu=True, scale_out=False)
    g3 = _mm_plain_scaled(h3, dis, W4.astype(bf))
    S3 = agg(g3, 16)
    out = _mm_final(S3, g3, dis, b4.reshape(1, -1), Wo.astype(bf),
                    bo.reshape(1, -1))
    return out[:N_NODES]


# speedup vs baseline: 4.8675x; 1.2285x over previous
import functools

import jax
import jax.numpy as jnp
from jax import lax
from jax.experimental import pallas as pl
from jax.experimental.pallas import tpu as pltpu
from jax.experimental.pallas import tpu_sc as plsc

N_NODES = 10000
N_PAD = 10240
E_EDGES = 160000
E_PAD = 163840
NC, NS = 2, 16
WIN = 128
ROWS_PER_TILE = N_PAD // NS
EDGES_PER_W32 = E_PAD // (NC * NS)
EDGES_PER_W16 = E_PAD // NS

_MESH = dict(core_axis_name="c", subcore_axis_name="s", num_cores=NC,
             num_subcores=NS)


def _make_deg_kernel():
    mesh = plsc.VectorSubcoreMesh(**_MESH)

    @functools.partial(
        pl.kernel,
        out_type=jax.ShapeDtypeStruct((NC, N_PAD, 128), jnp.float32),
        mesh=mesh,
        scratch_types=[
            pltpu.VMEM_SHARED((N_PAD, 128), jnp.float32),
            pltpu.VMEM((WIN, 128), jnp.float32),
            pltpu.VMEM((WIN,), jnp.int32),
        ],
    )
    def deg_k(colp, ones_hbm, zeros_hbm, out, slab, ones_v, cidx):
        c = lax.axis_index("c")
        s = lax.axis_index("s")
        wid = c * NS + s
        pltpu.sync_copy(zeros_hbm, slab.at[pl.ds(s * ROWS_PER_TILE,
                                                 ROWS_PER_TILE)])
        pltpu.sync_copy(ones_hbm, ones_v)
        plsc.subcore_barrier()

        def body(w, carry):
            base = wid * EDGES_PER_W32 + w * WIN
            pltpu.sync_copy(colp.at[pl.ds(base, WIN)], cidx)
            pltpu.sync_copy(ones_v, slab.at[cidx], add=True)
            return carry

        lax.fori_loop(0, EDGES_PER_W32 // WIN, body, 0)
        plsc.subcore_barrier()
        pltpu.sync_copy(slab.at[pl.ds(s * ROWS_PER_TILE, ROWS_PER_TILE)],
                        out.at[c, pl.ds(s * ROWS_PER_TILE, ROWS_PER_TILE)])

    return deg_k


NWIN = EDGES_PER_W16 // WIN
NWIN_H = NWIN // 2
SLAB_ROWS = N_PAD
TROWS = SLAB_ROWS // NS


def _make_agg_kernel(K):
    mesh = plsc.VectorSubcoreMesh(**_MESH)
    cpc = K // NC

    @functools.partial(
        pl.kernel,
        out_type=jax.ShapeDtypeStruct((K, N_PAD, 128), jnp.float32),
        mesh=mesh,
        scratch_types=[
            pltpu.VMEM_SHARED((SLAB_ROWS, 128), jnp.float32),
            pltpu.VMEM((WIN, 128), jnp.float32),
            pltpu.VMEM((WIN, 128), jnp.float32),
            [pltpu.VMEM((WIN,), jnp.int32)] * 4,
            [pltpu.VMEM((WIN,), jnp.int32)] * 4,
            [pltpu.SemaphoreType.DMA] * 4,
            [pltpu.SemaphoreType.DMA] * 2,
            [pltpu.SemaphoreType.DMA] * 2,
        ],
    )
    def agg_k(g2df, gidxf, colf, zeros_hbm, out, slab, rows0, rows1,
              gbuf, cbuf, isem, gsem, ssem):
        cid = lax.axis_index("c")
        s = lax.axis_index("s")
        rows = (rows0, rows1)

        def idx_start(w, q, ch):
            gbase = (ch * NS + s) * EDGES_PER_W16 + w * WIN
            cbase = s * EDGES_PER_W16 + w * WIN
            pltpu.async_copy(gidxf.at[pl.ds(gbase, WIN)], gbuf[q],
                             isem[q])
            pltpu.async_copy(colf.at[pl.ds(cbase, WIN)], cbuf[q],
                             isem[q])

        def idx_wait(q):
            pltpu.make_async_copy(colf.at[pl.ds(0, WIN)], gbuf[q],
                                  isem[q]).wait()
            pltpu.make_async_copy(colf.at[pl.ds(0, WIN)], cbuf[q],
                                  isem[q]).wait()

        def gather_start(q, p):
            pltpu.async_copy(g2df.at[gbuf[q]], rows[p], gsem[p])

        def gather_wait(q, p):
            pltpu.make_async_copy(g2df.at[gbuf[q]], rows[p],
                                  gsem[p]).wait()

        def scatter_start(q, p):
            pltpu.async_copy(rows[p], slab.at[cbuf[q]], ssem[p],
                             add=True)

        def scatter_wait(q, p):
            pltpu.make_async_copy(rows[p], slab.at[cbuf[q]],
                                  ssem[p]).wait()

        def step(w, q, ch, do_idx, do_gather):
            p = q % 2
            gather_wait(q, p)
            scatter_start(q, p)
            scatter_wait(q, p)
            if do_idx:
                idx_start(w + 4, q, ch)
            if do_gather:
                q2 = (q + 2) % 4
                idx_wait(q2)
                gather_start(q2, p)

        for cl in range(cpc):
            ch = cid * cpc + cl
            pltpu.sync_copy(zeros_hbm, slab.at[pl.ds(s * TROWS, TROWS)])
            plsc.subcore_barrier()

            for w in range(4):
                idx_start(w, w, ch)
            for w in range(2):
                idx_wait(w)
                gather_start(w, w)

            def body(j, carry):
                for qq in range(4):
                    step(4 * j + qq, qq, ch, True, True)
                return carry

            lax.fori_loop(0, NWIN // 4 - 2, body, 0)
            for w in range(NWIN - 8, NWIN):
                step(w, w % 4, ch, w + 4 < NWIN, w + 2 < NWIN)
            plsc.subcore_barrier()
            pltpu.sync_copy(
                slab.at[pl.ds(s * TROWS, TROWS)],
                out.at[ch, pl.ds(s * TROWS, TROWS)])
            plsc.subcore_barrier()

    return agg_k


_BM = 256


def _prep_body(h0_ref, h1_ref, x_ref, dis_ref, g0_ref):
    deg128 = h0_ref[...] + h1_ref[...] + 1.0
    dis_ref[...] = lax.rsqrt(deg128)
    dis = lax.rsqrt(deg128[:, :1])
    g0_ref[...] = x_ref[...] * dis


def _prep(h0, h1, x_p):
    grid = (N_PAD // _BM,)
    return pl.pallas_call(
        _prep_body,
        grid=grid,
        in_specs=[
            pl.BlockSpec((_BM, 128), lambda i: (i, 0)),
            pl.BlockSpec((_BM, 128), lambda i: (i, 0)),
            pl.BlockSpec((_BM, 256), lambda i: (i, 0)),
        ],
        out_specs=[
            pl.BlockSpec((_BM, 128), lambda i: (i, 0)),
            pl.BlockSpec((_BM, 256), lambda i: (i, 0)),
        ],
        out_shape=[
            jax.ShapeDtypeStruct((N_PAD, 128), jnp.float32),
            jax.ShapeDtypeStruct((N_PAD, 256), jnp.float32),
        ],
    )(h0, h1, x_p)


def _mm_layer(S, g, dis, W, b, relu, scale_out, bm=512, bn=1024):
    K = S.shape[0]
    dout = W.shape[1]
    bn = min(bn, dout)
    kb = 4 if K % 4 == 0 else K
    grid = (N_PAD // bm, dout // bn, K // kb)

    def body(s_ref, g_ref, dis_ref, w_ref, b_ref, out_ref):
        k = pl.program_id(2)
        nk = pl.num_programs(2)
        d = dis_ref[...]
        a = jnp.concatenate(
            [(s_ref[c] + g_ref[:, c * 128:(c + 1) * 128]) * d
             for c in range(kb)], axis=1)
        part = jnp.dot(a.astype(jnp.bfloat16), w_ref[...],
                       preferred_element_type=jnp.float32)

        @pl.when(k == 0)
        def _():
            out_ref[...] = part

        @pl.when(k > 0)
        def _():
            out_ref[...] += part

        @pl.when(k == nk - 1)
        def _():
            r = out_ref[...] + b_ref[...]
            if relu:
                r = jnp.maximum(r, 0.0)
            if scale_out:
                r = r * jnp.broadcast_to(dis_ref[:, :1], r.shape)
            out_ref[...] = r

    return pl.pallas_call(
        body,
        grid=grid,
        in_specs=[
            pl.BlockSpec((kb, bm, 128), lambda i, j, k: (k, i, 0)),
            pl.BlockSpec((bm, kb * 128), lambda i, j, k: (i, k)),
            pl.BlockSpec((bm, 128), lambda i, j, k: (i, 0)),
            pl.BlockSpec((kb * 128, bn), lambda i, j, k: (k, j)),
            pl.BlockSpec((1, bn), lambda i, j, k: (0, j)),
        ],
        out_specs=pl.BlockSpec((bm, bn), lambda i, j, k: (i, j)),
        out_shape=jax.ShapeDtypeStruct((N_PAD, dout), jnp.float32),
        compiler_params=pltpu.CompilerParams(
            dimension_semantics=("parallel", "parallel", "arbitrary")),
    )(S, g, dis, W, b)


def _mm_plain_scaled(A, dis, W, bm=512, bn=1024, bk=512):
    din, dout = W.shape
    grid = (N_PAD // bm, dout // bn, din // bk)

    def body(a_ref, dis_ref, w_ref, out_ref):
        k = pl.program_id(2)
        nk = pl.num_programs(2)
        part = jnp.dot(a_ref[...].astype(jnp.bfloat16), w_ref[...],
                       preferred_element_type=jnp.float32)

        @pl.when(k == 0)
        def _():
            out_ref[...] = part

        @pl.when(k > 0)
        def _():
            out_ref[...] += part

        @pl.when(k == nk - 1)
        def _():
            out_ref[...] = out_ref[...] * jnp.broadcast_to(
                dis_ref[:, :1], out_ref.shape)

    return pl.pallas_call(
        body,
        grid=grid,
        in_specs=[
            pl.BlockSpec((bm, bk), lambda i, j, k: (i, k)),
            pl.BlockSpec((bm, 128), lambda i, j, k: (i, 0)),
            pl.BlockSpec((bk, bn), lambda i, j, k: (k, j)),
        ],
        out_specs=pl.BlockSpec((bm, bn), lambda i, j, k: (i, j)),
        out_shape=jax.ShapeDtypeStruct((N_PAD, dout), jnp.float32),
        compiler_params=pltpu.CompilerParams(
            dimension_semantics=("parallel", "parallel", "arbitrary")),
    )(A, dis, W)


def _mm_final(S, g, dis, b4, Wo, bo, bm=512):
    K = S.shape[0]
    grid = (N_PAD // bm, K)

    def body(s_ref, g_ref, dis_ref, b4_ref, wo_ref, bo_ref, out_ref):
        k = pl.program_id(1)
        nk = pl.num_programs(1)
        h = (s_ref[0] + g_ref[...]) * dis_ref[...] + b4_ref[...]
        h = jnp.maximum(h, 0.0)
        part = jnp.dot(h.astype(jnp.bfloat16), wo_ref[...],
                       preferred_element_type=jnp.float32)

        @pl.when(k == 0)
        def _():
            out_ref[...] = part

        @pl.when(k > 0)
        def _():
            out_ref[...] += part

        @pl.when(k == nk - 1)
        def _():
            out_ref[...] += bo_ref[...]

    return pl.pallas_call(
        body,
        grid=grid,
        in_specs=[
            pl.BlockSpec((1, bm, 128), lambda i, k: (k, i, 0)),
            pl.BlockSpec((bm, 128), lambda i, k: (i, k)),
            pl.BlockSpec((bm, 128), lambda i, k: (i, 0)),
            pl.BlockSpec((1, 128), lambda i, k: (0, k)),
            pl.BlockSpec((128, 128), lambda i, k: (k, 0)),
            pl.BlockSpec((1, 128), lambda i, k: (0, 0)),
        ],
        out_specs=pl.BlockSpec((bm, 128), lambda i, k: (i, 0)),
        out_shape=jax.ShapeDtypeStruct((N_PAD, 128), jnp.float32),
        compiler_params=pltpu.CompilerParams(
            dimension_semantics=("parallel", "arbitrary")),
    )(S, g, dis, b4, Wo, bo)


_deg_kernel = _make_deg_kernel()
_agg_kernels = {K: _make_agg_kernel(K) for K in (2, 8, 16)}


def kernel(x, edge_index, W1, b1, W2, b2, W3, b3, W4, b4, Wo, bo):
    row = edge_index[0].astype(jnp.int32)
    col = edge_index[1].astype(jnp.int32)
    npad = E_PAD - E_EDGES
    pad_i = jnp.arange(npad, dtype=jnp.int32)
    rowp = jnp.concatenate([row, pad_i % N_NODES])
    colp = jnp.concatenate([col, N_NODES + pad_i % (SLAB_ROWS - N_NODES)])

    ones128 = jnp.ones((WIN, 128), jnp.float32)
    zeros128 = jnp.zeros((ROWS_PER_TILE, 128), jnp.float32)

    hist = _deg_kernel(colp, ones128, zeros128)
    x_p = jnp.pad(x, ((0, N_PAD - N_NODES), (0, 0)))
    dis, g0 = _prep(hist[0], hist[1], x_p)

    roww = rowp.reshape(NS, NWIN, WIN)

    def agg(g, K):
        gidxf = (roww[None] * K
                 + jnp.arange(K, dtype=jnp.int32)[:, None, None, None]
                 ).reshape(-1)
        return _agg_kernels[K](g.reshape(N_PAD * K, 128), gidxf, colp,
                               zeros128)

    bf = jnp.bfloat16
    S0 = agg(g0, 2)
    g1 = _mm_layer(S0, g0, dis, W1.astype(bf), b1.reshape(1, -1),
                   relu=True, scale_out=True)
    S1 = agg(g1, 8)
    g2 = _mm_layer(S1, g1, dis, W2.astype(bf), b2.reshape(1, -1),
                   relu=True, scale_out=True)
    S2 = agg(g2, 16)
    h3 = _mm_layer(S2, g2, dis, W3.astype(bf), b3.reshape(1, -1),
                   relu=True, scale_out=False)
    g3 = _mm_plain_scaled(h3, dis, W4.astype(bf))
    S3 = agg(g3, 16)
    out = _mm_final(S3, g3, dis, b4.reshape(1, -1), Wo.astype(bf),
                    bo.reshape(1, -1))
    return out[:N_NODES]


# speedup vs baseline: 5.0351x; 1.0344x over previous
import functools

import jax
import jax.numpy as jnp
from jax import lax
from jax.experimental import pallas as pl
from jax.experimental.pallas import tpu as pltpu
from jax.experimental.pallas import tpu_sc as plsc

N_NODES = 10000
N_PAD = 10240
E_EDGES = 160000
E_PAD = 163840
NC, NS = 2, 16
WIN = 128
ROWS_PER_TILE = N_PAD // NS
EDGES_PER_W32 = E_PAD // (NC * NS)
EDGES_PER_W16 = E_PAD // NS

_MESH = dict(core_axis_name="c", subcore_axis_name="s", num_cores=NC,
             num_subcores=NS)


def _make_deg_kernel():
    mesh = plsc.VectorSubcoreMesh(**_MESH)

    @functools.partial(
        pl.kernel,
        out_type=jax.ShapeDtypeStruct((NC, N_PAD, 128), jnp.float32),
        mesh=mesh,
        scratch_types=[
            pltpu.VMEM_SHARED((N_PAD, 128), jnp.float32),
            pltpu.VMEM((WIN, 128), jnp.float32),
            pltpu.VMEM((WIN,), jnp.int32),
        ],
    )
    def deg_k(colp, ones_hbm, zeros_hbm, out, slab, ones_v, cidx):
        c = lax.axis_index("c")
        s = lax.axis_index("s")
        wid = c * NS + s
        pltpu.sync_copy(zeros_hbm, slab.at[pl.ds(s * ROWS_PER_TILE,
                                                 ROWS_PER_TILE)])
        pltpu.sync_copy(ones_hbm, ones_v)
        plsc.subcore_barrier()

        def body(w, carry):
            base = wid * EDGES_PER_W32 + w * WIN
            pltpu.sync_copy(colp.at[pl.ds(base, WIN)], cidx)
            pltpu.sync_copy(ones_v, slab.at[cidx], add=True)
            return carry

        lax.fori_loop(0, EDGES_PER_W32 // WIN, body, 0)
        plsc.subcore_barrier()
        pltpu.sync_copy(slab.at[pl.ds(s * ROWS_PER_TILE, ROWS_PER_TILE)],
                        out.at[c, pl.ds(s * ROWS_PER_TILE, ROWS_PER_TILE)])

    return deg_k


NWIN = EDGES_PER_W16 // WIN
NWIN_H = NWIN // 2
SLAB_ROWS = N_PAD
TROWS = SLAB_ROWS // NS


def _make_agg_kernel(K):
    mesh = plsc.VectorSubcoreMesh(**_MESH)
    cpc = K // NC

    @functools.partial(
        pl.kernel,
        out_type=jax.ShapeDtypeStruct((K, N_PAD, 128), jnp.float32),
        mesh=mesh,
        scratch_types=[
            pltpu.VMEM_SHARED((SLAB_ROWS, 128), jnp.float32),
            pltpu.VMEM((WIN, 128), jnp.float32),
            pltpu.VMEM((WIN, 128), jnp.float32),
            [pltpu.VMEM((WIN,), jnp.int32)] * 4,
            [pltpu.VMEM((WIN,), jnp.int32)] * 4,
            [pltpu.SemaphoreType.DMA] * 4,
            [pltpu.SemaphoreType.DMA] * 2,
            [pltpu.SemaphoreType.DMA] * 2,
        ],
    )
    def agg_k(g2df, gidxf, colf, zeros_hbm, out, slab, rows0, rows1,
              gbuf, cbuf, isem, gsem, ssem):
        cid = lax.axis_index("c")
        s = lax.axis_index("s")
        rows = (rows0, rows1)

        def idx_start(w, q, ch):
            gbase = (ch * NS + s) * EDGES_PER_W16 + w * WIN
            cbase = s * EDGES_PER_W16 + w * WIN
            pltpu.async_copy(gidxf.at[pl.ds(gbase, WIN)], gbuf[q],
                             isem[q])
            pltpu.async_copy(colf.at[pl.ds(cbase, WIN)], cbuf[q],
                             isem[q])

        def idx_wait(q):
            pltpu.make_async_copy(colf.at[pl.ds(0, WIN)], gbuf[q],
                                  isem[q]).wait()
            pltpu.make_async_copy(colf.at[pl.ds(0, WIN)], cbuf[q],
                                  isem[q]).wait()

        def gather_start(q, p):
            pltpu.async_copy(g2df.at[gbuf[q]], rows[p], gsem[p])

        def gather_wait(q, p):
            pltpu.make_async_copy(g2df.at[gbuf[q]], rows[p],
                                  gsem[p]).wait()

        def scatter_start(q, p):
            pltpu.async_copy(rows[p], slab.at[cbuf[q]], ssem[p],
                             add=True)

        def scatter_wait(q, p):
            pltpu.make_async_copy(rows[p], slab.at[cbuf[q]],
                                  ssem[p]).wait()

        def step(w, q, ch, do_idx, do_gather):
            p = q % 2
            gather_wait(q, p)
            scatter_start(q, p)
            scatter_wait(q, p)
            if do_idx:
                idx_start(w + 4, q, ch)
            if do_gather:
                q2 = (q + 2) % 4
                idx_wait(q2)
                gather_start(q2, p)

        for cl in range(cpc):
            ch = cid * cpc + cl
            pltpu.sync_copy(zeros_hbm, slab.at[pl.ds(s * TROWS, TROWS)])
            plsc.subcore_barrier()

            for w in range(4):
                idx_start(w, w, ch)
            for w in range(2):
                idx_wait(w)
                gather_start(w, w)

            def body(j, carry):
                for qq in range(4):
                    step(4 * j + qq, qq, ch, True, True)
                return carry

            lax.fori_loop(0, NWIN // 4 - 2, body, 0)
            for w in range(NWIN - 8, NWIN):
                step(w, w % 4, ch, w + 4 < NWIN, w + 2 < NWIN)
            plsc.subcore_barrier()
            pltpu.sync_copy(
                slab.at[pl.ds(s * TROWS, TROWS)],
                out.at[ch, pl.ds(s * TROWS, TROWS)])
            plsc.subcore_barrier()

    return agg_k


_BM = 256


def _prep_body(h0_ref, h1_ref, x_ref, dis_ref, g0_ref):
    deg128 = h0_ref[...] + h1_ref[...] + 1.0
    dis_ref[...] = lax.rsqrt(deg128)
    dis = lax.rsqrt(deg128[:, :1])
    g0_ref[...] = x_ref[...] * dis


def _prep(h0, h1, x_p):
    grid = (N_PAD // _BM,)
    return pl.pallas_call(
        _prep_body,
        grid=grid,
        in_specs=[
            pl.BlockSpec((_BM, 128), lambda i: (i, 0)),
            pl.BlockSpec((_BM, 128), lambda i: (i, 0)),
            pl.BlockSpec((_BM, 256), lambda i: (i, 0)),
        ],
        out_specs=[
            pl.BlockSpec((_BM, 128), lambda i: (i, 0)),
            pl.BlockSpec((_BM, 256), lambda i: (i, 0)),
        ],
        out_shape=[
            jax.ShapeDtypeStruct((N_PAD, 128), jnp.float32),
            jax.ShapeDtypeStruct((N_PAD, 256), jnp.float32),
        ],
    )(h0, h1, x_p)


def _mm_layer(S, g, dis, W, b, relu, scale_out, bm=512, bn=1024):
    K = S.shape[0]
    dout = W.shape[1]
    bn = min(bn, dout)
    kb = 4 if K % 4 == 0 else K
    grid = (N_PAD // bm, dout // bn, K // kb)

    def body(s_ref, g_ref, dis_ref, w_ref, b_ref, out_ref):
        k = pl.program_id(2)
        nk = pl.num_programs(2)
        d = dis_ref[...]
        a = jnp.concatenate(
            [(s_ref[c] + g_ref[:, c * 128:(c + 1) * 128]) * d
             for c in range(kb)], axis=1)
        part = jnp.dot(a.astype(jnp.bfloat16), w_ref[...],
                       preferred_element_type=jnp.float32)

        @pl.when(k == 0)
        def _():
            out_ref[...] = part

        @pl.when(k > 0)
        def _():
            out_ref[...] += part

        @pl.when(k == nk - 1)
        def _():
            r = out_ref[...] + b_ref[...]
            if relu:
                r = jnp.maximum(r, 0.0)
            if scale_out:
                r = r * jnp.broadcast_to(dis_ref[:, :1], r.shape)
            out_ref[...] = r

    return pl.pallas_call(
        body,
        grid=grid,
        in_specs=[
            pl.BlockSpec((kb, bm, 128), lambda i, j, k: (k, i, 0)),
            pl.BlockSpec((bm, kb * 128), lambda i, j, k: (i, k)),
            pl.BlockSpec((bm, 128), lambda i, j, k: (i, 0)),
            pl.BlockSpec((kb * 128, bn), lambda i, j, k: (k, j)),
            pl.BlockSpec((1, bn), lambda i, j, k: (0, j)),
        ],
        out_specs=pl.BlockSpec((bm, bn), lambda i, j, k: (i, j)),
        out_shape=jax.ShapeDtypeStruct((N_PAD, dout), jnp.float32),
        compiler_params=pltpu.CompilerParams(
            dimension_semantics=("parallel", "parallel", "arbitrary")),
    )(S, g, dis, W, b)


def _mm_plain_scaled(A, dis, W, bm=512, bn=1024, bk=512):
    din, dout = W.shape
    grid = (N_PAD // bm, dout // bn, din // bk)

    def body(a_ref, dis_ref, w_ref, out_ref):
        k = pl.program_id(2)
        nk = pl.num_programs(2)
        part = jnp.dot(a_ref[...].astype(jnp.bfloat16), w_ref[...],
                       preferred_element_type=jnp.float32)

        @pl.when(k == 0)
        def _():
            out_ref[...] = part

        @pl.when(k > 0)
        def _():
            out_ref[...] += part

        @pl.when(k == nk - 1)
        def _():
            out_ref[...] = out_ref[...] * jnp.broadcast_to(
                dis_ref[:, :1], out_ref.shape)

    return pl.pallas_call(
        body,
        grid=grid,
        in_specs=[
            pl.BlockSpec((bm, bk), lambda i, j, k: (i, k)),
            pl.BlockSpec((bm, 128), lambda i, j, k: (i, 0)),
            pl.BlockSpec((bk, bn), lambda i, j, k: (k, j)),
        ],
        out_specs=pl.BlockSpec((bm, bn), lambda i, j, k: (i, j)),
        out_shape=jax.ShapeDtypeStruct((N_PAD, dout), jnp.float32),
        compiler_params=pltpu.CompilerParams(
            dimension_semantics=("parallel", "parallel", "arbitrary")),
    )(A, dis, W)


def _mm_final(S, g, dis, b4, Wo, bo, bm=512, kb=4):
    K = S.shape[0]
    grid = (N_PAD // bm, K // kb)

    def body(s_ref, g_ref, dis_ref, b4_ref, wo_ref, bo_ref, out_ref):
        k = pl.program_id(1)
        nk = pl.num_programs(1)
        d = dis_ref[...]
        h = jnp.concatenate(
            [(s_ref[c] + g_ref[:, c * 128:(c + 1) * 128]) * d
             for c in range(kb)], axis=1) + b4_ref[...]
        h = jnp.maximum(h, 0.0)
        part = jnp.dot(h.astype(jnp.bfloat16), wo_ref[...],
                       preferred_element_type=jnp.float32)

        @pl.when(k == 0)
        def _():
            out_ref[...] = part

        @pl.when(k > 0)
        def _():
            out_ref[...] += part

        @pl.when(k == nk - 1)
        def _():
            out_ref[...] += bo_ref[...]

    return pl.pallas_call(
        body,
        grid=grid,
        in_specs=[
            pl.BlockSpec((kb, bm, 128), lambda i, k: (k, i, 0)),
            pl.BlockSpec((bm, kb * 128), lambda i, k: (i, k)),
            pl.BlockSpec((bm, 128), lambda i, k: (i, 0)),
            pl.BlockSpec((1, kb * 128), lambda i, k: (0, k)),
            pl.BlockSpec((kb * 128, 128), lambda i, k: (k, 0)),
            pl.BlockSpec((1, 128), lambda i, k: (0, 0)),
        ],
        out_specs=pl.BlockSpec((bm, 128), lambda i, k: (i, 0)),
        out_shape=jax.ShapeDtypeStruct((N_PAD, 128), jnp.float32),
        compiler_params=pltpu.CompilerParams(
            dimension_semantics=("parallel", "arbitrary")),
    )(S, g, dis, b4, Wo, bo)


_deg_kernel = _make_deg_kernel()
_agg_kernels = {K: _make_agg_kernel(K) for K in (2, 8, 16)}


def kernel(x, edge_index, W1, b1, W2, b2, W3, b3, W4, b4, Wo, bo):
    row = edge_index[0].astype(jnp.int32)
    col = edge_index[1].astype(jnp.int32)
    npad = E_PAD - E_EDGES
    pad_i = jnp.arange(npad, dtype=jnp.int32)
    rowp = jnp.concatenate([row, pad_i % N_NODES])
    colp = jnp.concatenate([col, N_NODES + pad_i % (SLAB_ROWS - N_NODES)])

    ones128 = jnp.ones((WIN, 128), jnp.float32)
    zeros128 = jnp.zeros((ROWS_PER_TILE, 128), jnp.float32)

    hist = _deg_kernel(colp, ones128, zeros128)
    x_p = jnp.pad(x, ((0, N_PAD - N_NODES), (0, 0)))
    dis, g0 = _prep(hist[0], hist[1], x_p)

    roww = rowp.reshape(NS, NWIN, WIN)

    def agg(g, K):
        gidxf = (roww[None] * K
                 + jnp.arange(K, dtype=jnp.int32)[:, None, None, None]
                 ).reshape(-1)
        return _agg_kernels[K](g.reshape(N_PAD * K, 128), gidxf, colp,
                               zeros128)

    bf = jnp.bfloat16
    S0 = agg(g0, 2)
    g1 = _mm_layer(S0, g0, dis, W1.astype(bf), b1.reshape(1, -1),
                   relu=True, scale_out=True)
    S1 = agg(g1, 8)
    g2 = _mm_layer(S1, g1, dis, W2.astype(bf), b2.reshape(1, -1),
                   relu=True, scale_out=True)
    S2 = agg(g2, 16)
    h3 = _mm_layer(S2, g2, dis, W3.astype(bf), b3.reshape(1, -1),
                   relu=True, scale_out=False)
    g3 = _mm_plain_scaled(h3, dis, W4.astype(bf))
    S3 = agg(g3, 16)
    out = _mm_final(S3, g3, dis, b4.reshape(1, -1), Wo.astype(bf),
                    bo.reshape(1, -1))
    return out[:N_NODES]
